# Initial kernel scaffold; baseline (speedup 1.0000x reference)
import jax
import jax.numpy as jnp
from jax.experimental import pallas as pl


def kernel(x, edge_index, edge_type, comp, bases, root, conv_bias, w_global, b_global, w_sense, b_sense):
    raise NotImplementedError("write your pallas kernel here")


# speedup vs baseline: 100.1249x; 100.1249x over previous
import jax
import jax.numpy as jnp
from jax import lax
from jax.experimental import pallas as pl
from jax.experimental.pallas import tpu as pltpu
from jax.experimental.pallas import tpu_sc as plsc

_N = 10000
_E = 320000
_D = 128
_R = 5
_NB = 5
_OUT_G = 10000
_OUT_S = 25000

_NC = 2
_NS = 16
_NW = _NC * _NS
_EW = _E // _NW
_CH = 16
_NCH = _EW // _CH
_AR = 8


def _sc_edge_reduce(edge_flat, edge_type, x, sums_out, counts_out,
                    dst_v, src16, et16, idx_ref, etidx_ref, rows_v, ones_v,
                    acc_tmp, cnt_tmp, acc_sh, cnt_sh, sem):
    c = lax.axis_index("c")
    s = lax.axis_index("s")
    w = s * _NC + c
    base = w * _EW

    pltpu.sync_copy(edge_flat.at[pl.ds(_E + base, _EW)], dst_v)

    for j in range(_CH):
        for kk in range(_D // 16):
            ones_v[j, pl.ds(kk * 16, 16)] = jnp.ones((16,), jnp.float32)
    for rr in range(_AR):
        for kk in range(_D // 16):
            acc_tmp[rr, pl.ds(kk * 16, 16)] = jnp.zeros((16,), jnp.float32)
        for kk in range(_D // 16):
            cnt_tmp[rr, pl.ds(kk * 16, 16)] = jnp.zeros((16,), jnp.float32)

    @pl.when(s == 0)
    def _():
        pltpu.sync_copy(acc_tmp, acc_sh)
        pltpu.sync_copy(cnt_tmp, cnt_sh)

    plsc.subcore_barrier()

    def chunk(i, carry):
        dstv = dst_v[pl.ds(i * _CH, _CH)]
        mask = dstv == 0
        nhit = jnp.max(plsc.all_reduce_population_count(mask))

        @pl.when(nhit > 0)
        def _():
            off = base + i * _CH
            pltpu.sync_copy(edge_flat.at[pl.ds(off, _CH)], src16)
            pltpu.sync_copy(edge_type.at[pl.ds(off, _CH)], et16)
            idx_ref[...] = jnp.where(mask, src16[...], 0)
            etidx_ref[...] = jnp.where(mask, et16[...], _R)
            pltpu.async_copy(x.at[idx_ref], rows_v, sem).wait()
            pltpu.sync_copy(rows_v, acc_sh.at[etidx_ref], add=True)
            pltpu.sync_copy(ones_v, cnt_sh.at[etidx_ref], add=True)

        return carry

    lax.fori_loop(0, _NCH, chunk, 0)
    plsc.subcore_barrier()

    @pl.when(s == 0)
    def _():
        pltpu.sync_copy(acc_sh, acc_tmp)
        pltpu.sync_copy(acc_tmp, sums_out.at[c])
        pltpu.sync_copy(cnt_sh, cnt_tmp)
        pltpu.sync_copy(cnt_tmp, counts_out.at[c])


def _tc_dense(sums_ref, counts_ref, comp_ref, bases_ref, root_ref, bias_ref,
              x0_ref, wg_ref, bg_ref, ws_ref, bs_ref, outg_ref, outs_ref):
    ssum = sums_ref[0] + sums_ref[1]
    call = counts_ref[0] + counts_ref[1]
    cnt = jnp.max(call, axis=1, keepdims=True)
    mean = ssum / jnp.maximum(cnt, 1.0)
    mfive = mean[0:_R]
    mb = lax.dot_general(comp_ref[...], mfive, (((0,), (0,)), ((), ())),
                         preferred_element_type=jnp.float32)
    agg = jnp.zeros((1, _D), jnp.float32)
    for b in range(_NB):
        agg = agg + jnp.dot(mb[b:b + 1, :], bases_ref[b],
                            preferred_element_type=jnp.float32)
    v = agg + jnp.dot(x0_ref[...], root_ref[...],
                      preferred_element_type=jnp.float32) + bias_ref[...]
    v = jnp.maximum(v, 0.0)

    for w_ref, b_ref, out_ref in ((wg_ref, bg_ref, outg_ref),
                                  (ws_ref, bs_ref, outs_ref)):
        lg = lax.dot_general(v, w_ref[...], (((1,), (1,)), ((), ())),
                             preferred_element_type=jnp.float32) + b_ref[...]
        m = jnp.max(lg)
        lse = m + jnp.log(jnp.sum(jnp.exp(lg - m)))
        out_ref[...] = lg - lse


def _dense_heads(sums, counts, comp, bases, root, conv_bias,
                 x0, w_global, b_global, w_sense, b_sense):
    return pl.pallas_call(
        _tc_dense,
        out_shape=(jax.ShapeDtypeStruct((1, _OUT_G), jnp.float32),
                   jax.ShapeDtypeStruct((1, _OUT_S), jnp.float32)),
    )(sums, counts, comp, bases, root, conv_bias.reshape(1, _D), x0,
      w_global, b_global.reshape(1, _OUT_G), w_sense, b_sense.reshape(1, _OUT_S))


def _edge_stats(x, edge_index, edge_type):
    mesh = plsc.VectorSubcoreMesh(core_axis_name="c", subcore_axis_name="s")
    f32 = jnp.float32
    return pl.kernel(
        _sc_edge_reduce,
        out_type=(jax.ShapeDtypeStruct((_NC, _AR, _D), f32),
                  jax.ShapeDtypeStruct((_NC, _AR, _D), f32)),
        mesh=mesh,
        scratch_types=[
            pltpu.VMEM((_EW,), jnp.int32),
            pltpu.VMEM((_CH,), jnp.int32),
            pltpu.VMEM((_CH,), jnp.int32),
            pltpu.VMEM((_CH,), jnp.int32),
            pltpu.VMEM((_CH,), jnp.int32),
            pltpu.VMEM((_CH, _D), f32),
            pltpu.VMEM((_CH, _D), f32),
            pltpu.VMEM((_AR, _D), f32),
            pltpu.VMEM((_AR, _D), f32),
            pltpu.VMEM_SHARED((_AR, _D), f32),
            pltpu.VMEM_SHARED((_AR, _D), f32),
            pltpu.SemaphoreType.DMA,
        ],
        compiler_params=pltpu.CompilerParams(needs_layout_passes=False),
    )(edge_index.reshape(2 * _E), edge_type, x)


def kernel(x, edge_index, edge_type, comp, bases, root, conv_bias,
           w_global, b_global, w_sense, b_sense):
    sums, counts = _edge_stats(x, edge_index, edge_type)
    x0 = lax.slice(x, (0, 0), (1, _D))
    outg, outs = _dense_heads(sums, counts, comp, bases, root, conv_bias,
                              x0, w_global, b_global, w_sense, b_sense)
    return outg, outs


# speedup vs baseline: 123.3785x; 1.2322x over previous
import jax
import jax.numpy as jnp
from jax import lax
from jax.experimental import pallas as pl
from jax.experimental.pallas import tpu as pltpu
from jax.experimental.pallas import tpu_sc as plsc

_N = 10000
_E = 320000
_D = 128
_R = 5
_NB = 5
_OUT_G = 10000
_OUT_S = 25000

_NC = 2
_NS = 16
_NW = _NC * _NS
_EW = _E // _NW
_CH = 16
_NCH = _EW // _CH
_BC = 25
_AR = 8


def _sc_edge_reduce(edge_flat, edge_type, x, sums_out, counts_out,
                    dst_v, src16, et16, idx_ref, etidx_ref, rows_v, ones_v,
                    acc_tmp, cnt_tmp, acc_sh, cnt_sh, sem):
    c = lax.axis_index("c")
    s = lax.axis_index("s")
    w = s * _NC + c
    base = w * _EW

    pltpu.sync_copy(edge_flat.at[pl.ds(_E + base, _EW)], dst_v)

    for j in range(_CH):
        for kk in range(_D // 16):
            ones_v[j, pl.ds(kk * 16, 16)] = jnp.ones((16,), jnp.float32)
    for rr in range(_AR):
        for kk in range(_D // 16):
            acc_tmp[rr, pl.ds(kk * 16, 16)] = jnp.zeros((16,), jnp.float32)
        for kk in range(_D // 16):
            cnt_tmp[rr, pl.ds(kk * 16, 16)] = jnp.zeros((16,), jnp.float32)

    @pl.when(s == 0)
    def _():
        pltpu.sync_copy(acc_tmp, acc_sh)
        pltpu.sync_copy(cnt_tmp, cnt_sh)

    plsc.subcore_barrier()

    def hit_chunk(i, carry):
        dstv = dst_v[pl.ds(i * _CH, _CH)]
        mask = dstv == 0
        nhit = jnp.max(plsc.all_reduce_population_count(mask))

        @pl.when(nhit > 0)
        def _():
            off = base + i * _CH
            pltpu.sync_copy(edge_flat.at[pl.ds(off, _CH)], src16)
            pltpu.sync_copy(edge_type.at[pl.ds(off, _CH)], et16)
            idx_ref[...] = jnp.where(mask, src16[...], 0)
            etidx_ref[...] = jnp.where(mask, et16[...], _R)
            pltpu.async_copy(x.at[idx_ref], rows_v, sem).wait()
            pltpu.sync_copy(rows_v, acc_sh.at[etidx_ref], add=True)
            pltpu.sync_copy(ones_v, cnt_sh.at[etidx_ref], add=True)

        return carry

    def block(ib, carry):
        b0 = ib * _BC
        mind = dst_v[pl.ds(b0 * _CH, _CH)]
        for j in range(1, _BC):
            mind = jnp.minimum(mind, dst_v[pl.ds((b0 + j) * _CH, _CH)])
        nhit = jnp.max(plsc.all_reduce_population_count(mind == 0))

        @pl.when(nhit > 0)
        def _():
            lax.fori_loop(b0, b0 + _BC, hit_chunk, 0)

        return carry

    lax.fori_loop(0, _NCH // _BC, block, 0)
    plsc.subcore_barrier()

    @pl.when(s == 0)
    def _():
        pltpu.sync_copy(acc_sh, acc_tmp)
        pltpu.sync_copy(acc_tmp, sums_out.at[c])
        pltpu.sync_copy(cnt_sh, cnt_tmp)
        pltpu.sync_copy(cnt_tmp, counts_out.at[c])


def _tc_dense(sums_ref, counts_ref, comp_ref, bases_ref, root_ref, bias_ref,
              x0_ref, wg_ref, bg_ref, ws_ref, bs_ref, outg_ref, outs_ref):
    ssum = sums_ref[0] + sums_ref[1]
    call = counts_ref[0] + counts_ref[1]
    cnt = jnp.max(call, axis=1, keepdims=True)
    mean = ssum / jnp.maximum(cnt, 1.0)
    mfive = mean[0:_R]
    mb = lax.dot_general(comp_ref[...], mfive, (((0,), (0,)), ((), ())),
                         preferred_element_type=jnp.float32)
    agg = jnp.zeros((1, _D), jnp.float32)
    for b in range(_NB):
        agg = agg + jnp.dot(mb[b:b + 1, :], bases_ref[b],
                            preferred_element_type=jnp.float32)
    v = agg + jnp.dot(x0_ref[...], root_ref[...],
                      preferred_element_type=jnp.float32) + bias_ref[...]
    v = jnp.maximum(v, 0.0)

    for w_ref, b_ref, out_ref in ((wg_ref, bg_ref, outg_ref),
                                  (ws_ref, bs_ref, outs_ref)):
        lg = lax.dot_general(v, w_ref[...], (((1,), (1,)), ((), ())),
                             preferred_element_type=jnp.float32) + b_ref[...]
        m = jnp.max(lg)
        lse = m + jnp.log(jnp.sum(jnp.exp(lg - m)))
        out_ref[...] = lg - lse


def _dense_heads(sums, counts, comp, bases, root, conv_bias,
                 x0, w_global, b_global, w_sense, b_sense):
    return pl.pallas_call(
        _tc_dense,
        out_shape=(jax.ShapeDtypeStruct((1, _OUT_G), jnp.float32),
                   jax.ShapeDtypeStruct((1, _OUT_S), jnp.float32)),
    )(sums, counts, comp, bases, root, conv_bias.reshape(1, _D), x0,
      w_global, b_global.reshape(1, _OUT_G), w_sense, b_sense.reshape(1, _OUT_S))


def _edge_stats(x, edge_index, edge_type):
    mesh = plsc.VectorSubcoreMesh(core_axis_name="c", subcore_axis_name="s")
    f32 = jnp.float32
    return pl.kernel(
        _sc_edge_reduce,
        out_type=(jax.ShapeDtypeStruct((_NC, _AR, _D), f32),
                  jax.ShapeDtypeStruct((_NC, _AR, _D), f32)),
        mesh=mesh,
        scratch_types=[
            pltpu.VMEM((_EW,), jnp.int32),
            pltpu.VMEM((_CH,), jnp.int32),
            pltpu.VMEM((_CH,), jnp.int32),
            pltpu.VMEM((_CH,), jnp.int32),
            pltpu.VMEM((_CH,), jnp.int32),
            pltpu.VMEM((_CH, _D), f32),
            pltpu.VMEM((_CH, _D), f32),
            pltpu.VMEM((_AR, _D), f32),
            pltpu.VMEM((_AR, _D), f32),
            pltpu.VMEM_SHARED((_AR, _D), f32),
            pltpu.VMEM_SHARED((_AR, _D), f32),
            pltpu.SemaphoreType.DMA,
        ],
        compiler_params=pltpu.CompilerParams(needs_layout_passes=False),
    )(edge_index.reshape(2 * _E), edge_type, x)


def kernel(x, edge_index, edge_type, comp, bases, root, conv_bias,
           w_global, b_global, w_sense, b_sense):
    sums, counts = _edge_stats(x, edge_index, edge_type)
    x0 = lax.slice(x, (0, 0), (1, _D))
    outg, outs = _dense_heads(sums, counts, comp, bases, root, conv_bias,
                              x0, w_global, b_global, w_sense, b_sense)
    return outg, outs


# speedup vs baseline: 124.5164x; 1.0092x over previous
import jax
import jax.numpy as jnp
from jax import lax
from jax.experimental import pallas as pl
from jax.experimental.pallas import tpu as pltpu
from jax.experimental.pallas import tpu_sc as plsc

_N = 10000
_E = 320000
_D = 128
_R = 5
_NB = 5
_OUT_G = 10000
_OUT_S = 25000

_NC = 1
_NS = 16
_NW = _NC * _NS
_EW = _E // _NW
_CH = 16
_NCH = _EW // _CH
_BC = 25
_AR = 8


def _sc_edge_reduce(edge_flat, edge_type, x, sums_out, counts_out,
                    dst_v, src16, et16, idx_ref, etidx_ref, rows_v, ones_v,
                    acc_tmp, cnt_tmp, acc_sh, cnt_sh, sem):
    c = lax.axis_index("c")
    s = lax.axis_index("s")
    w = s * _NC + c
    base = w * _EW

    pltpu.sync_copy(edge_flat.at[pl.ds(_E + base, _EW)], dst_v)

    for j in range(_CH):
        for kk in range(_D // 16):
            ones_v[j, pl.ds(kk * 16, 16)] = jnp.ones((16,), jnp.float32)
    for rr in range(_AR):
        for kk in range(_D // 16):
            acc_tmp[rr, pl.ds(kk * 16, 16)] = jnp.zeros((16,), jnp.float32)
        for kk in range(_D // 16):
            cnt_tmp[rr, pl.ds(kk * 16, 16)] = jnp.zeros((16,), jnp.float32)

    @pl.when(s == 0)
    def _():
        pltpu.sync_copy(acc_tmp, acc_sh)
        pltpu.sync_copy(cnt_tmp, cnt_sh)

    plsc.subcore_barrier()

    def hit_chunk(i, carry):
        dstv = dst_v[pl.ds(i * _CH, _CH)]
        mask = dstv == 0
        nhit = jnp.max(plsc.all_reduce_population_count(mask))

        @pl.when(nhit > 0)
        def _():
            off = base + i * _CH
            pltpu.sync_copy(edge_flat.at[pl.ds(off, _CH)], src16)
            pltpu.sync_copy(edge_type.at[pl.ds(off, _CH)], et16)
            idx_ref[...] = jnp.where(mask, src16[...], 0)
            etidx_ref[...] = jnp.where(mask, et16[...], _R)
            pltpu.async_copy(x.at[idx_ref], rows_v, sem).wait()
            pltpu.sync_copy(rows_v, acc_sh.at[etidx_ref], add=True)
            pltpu.sync_copy(ones_v, cnt_sh.at[etidx_ref], add=True)

        return carry

    def block(ib, carry):
        b0 = ib * _BC
        mind = dst_v[pl.ds(b0 * _CH, _CH)]
        for j in range(1, _BC):
            mind = jnp.minimum(mind, dst_v[pl.ds((b0 + j) * _CH, _CH)])
        nhit = jnp.max(plsc.all_reduce_population_count(mind == 0))

        @pl.when(nhit > 0)
        def _():
            lax.fori_loop(b0, b0 + _BC, hit_chunk, 0)

        return carry

    lax.fori_loop(0, _NCH // _BC, block, 0)
    plsc.subcore_barrier()

    @pl.when(s == 0)
    def _():
        pltpu.sync_copy(acc_sh, acc_tmp)
        pltpu.sync_copy(acc_tmp, sums_out.at[c])
        pltpu.sync_copy(cnt_sh, cnt_tmp)
        pltpu.sync_copy(cnt_tmp, counts_out.at[c])


def _tc_dense(sums_ref, counts_ref, comp_ref, bases_ref, root_ref, bias_ref,
              x0_ref, wg_ref, bg_ref, ws_ref, bs_ref, outg_ref, outs_ref):
    ssum = jnp.sum(sums_ref[...], axis=0)
    call = jnp.sum(counts_ref[...], axis=0)
    cnt = jnp.max(call, axis=1, keepdims=True)
    mean = ssum / jnp.maximum(cnt, 1.0)
    mfive = mean[0:_R]
    mb = lax.dot_general(comp_ref[...], mfive, (((0,), (0,)), ((), ())),
                         preferred_element_type=jnp.float32)
    agg = jnp.zeros((1, _D), jnp.float32)
    for b in range(_NB):
        agg = agg + jnp.dot(mb[b:b + 1, :], bases_ref[b],
                            preferred_element_type=jnp.float32)
    v = agg + jnp.dot(x0_ref[...], root_ref[...],
                      preferred_element_type=jnp.float32) + bias_ref[...]
    v = jnp.maximum(v, 0.0)

    for w_ref, b_ref, out_ref in ((wg_ref, bg_ref, outg_ref),
                                  (ws_ref, bs_ref, outs_ref)):
        lg = lax.dot_general(v, w_ref[...], (((1,), (1,)), ((), ())),
                             preferred_element_type=jnp.float32) + b_ref[...]
        m = jnp.max(lg)
        lse = m + jnp.log(jnp.sum(jnp.exp(lg - m)))
        out_ref[...] = lg - lse


def _dense_heads(sums, counts, comp, bases, root, conv_bias,
                 x0, w_global, b_global, w_sense, b_sense):
    return pl.pallas_call(
        _tc_dense,
        out_shape=(jax.ShapeDtypeStruct((1, _OUT_G), jnp.float32),
                   jax.ShapeDtypeStruct((1, _OUT_S), jnp.float32)),
    )(sums, counts, comp, bases, root, conv_bias.reshape(1, _D), x0,
      w_global, b_global.reshape(1, _OUT_G), w_sense, b_sense.reshape(1, _OUT_S))


def _edge_stats(x, edge_index, edge_type):
    mesh = plsc.VectorSubcoreMesh(core_axis_name="c", subcore_axis_name="s",
                                  num_cores=_NC)
    f32 = jnp.float32
    return pl.kernel(
        _sc_edge_reduce,
        out_type=(jax.ShapeDtypeStruct((_NC, _AR, _D), f32),
                  jax.ShapeDtypeStruct((_NC, _AR, _D), f32)),
        mesh=mesh,
        scratch_types=[
            pltpu.VMEM((_EW,), jnp.int32),
            pltpu.VMEM((_CH,), jnp.int32),
            pltpu.VMEM((_CH,), jnp.int32),
            pltpu.VMEM((_CH,), jnp.int32),
            pltpu.VMEM((_CH,), jnp.int32),
            pltpu.VMEM((_CH, _D), f32),
            pltpu.VMEM((_CH, _D), f32),
            pltpu.VMEM((_AR, _D), f32),
            pltpu.VMEM((_AR, _D), f32),
            pltpu.VMEM_SHARED((_AR, _D), f32),
            pltpu.VMEM_SHARED((_AR, _D), f32),
            pltpu.SemaphoreType.DMA,
        ],
        compiler_params=pltpu.CompilerParams(needs_layout_passes=False),
    )(edge_index.reshape(2 * _E), edge_type, x)


def kernel(x, edge_index, edge_type, comp, bases, root, conv_bias,
           w_global, b_global, w_sense, b_sense):
    sums, counts = _edge_stats(x, edge_index, edge_type)
    x0 = lax.slice(x, (0, 0), (1, _D))
    outg, outs = _dense_heads(sums, counts, comp, bases, root, conv_bias,
                              x0, w_global, b_global, w_sense, b_sense)
    return outg, outs
